# baseline (speedup 1.0000x reference)
import functools

import jax
import jax.numpy as jnp
from jax import lax
from jax.experimental import pallas as pl
from jax.experimental.pallas import tpu as pltpu

_LOG2E = 1.4426950408889634
_LN2 = 0.6931471805599453
_INV_TEMPS = (1.0, 2.0)
_TM = 4096
_NSTREAM = 8
_NEG_BIG = -1.0e30


def _half_update(scores, m_scr, l_scr, th):
    chunks = [scores[:, g * 128:(g + 1) * 128] for g in range(th // 128)]
    tile_m = chunks[0]
    for c in chunks[1:]:
        tile_m = jnp.maximum(tile_m, c)
    m_prev = m_scr[...]
    m_new = jnp.maximum(m_prev, tile_m)
    part = jnp.exp2(chunks[0] - m_new)
    for c in chunks[1:]:
        part = part + jnp.exp2(c - m_new)
    alpha = jnp.exp2((m_prev - m_new).astype(jnp.float32))
    l_scr[...] = alpha * l_scr[...] + part.astype(jnp.float32)
    m_scr[...] = m_new


def _body(ref1_ref, pos1_ref, neg1_ref, ref2_ref, pos2_ref, neg2_ref,
          out_ref, r1s, r2s, *scratches, n, tm, num_mt, ns):
    mi = pl.program_id(0)
    th = tm // ns
    stats1 = [(scratches[2 * i], scratches[2 * i + 1]) for i in range(ns)]
    stats2 = [(scratches[2 * ns + 2 * i], scratches[2 * ns + 2 * i + 1])
              for i in range(ns)]

    @pl.when(mi == 0)
    def _init():
        r1s[...] = (ref1_ref[...] * jnp.float32(_INV_TEMPS[0] * _LOG2E)
                    ).astype(jnp.bfloat16)
        r2s[...] = (ref2_ref[...] * jnp.float32(_INV_TEMPS[1] * _LOG2E)
                    ).astype(jnp.bfloat16)
        for m_scr, l_scr in stats1 + stats2:
            m_scr[...] = jnp.full(m_scr.shape, _NEG_BIG, dtype=jnp.bfloat16)
            l_scr[...] = jnp.zeros(l_scr.shape, dtype=jnp.float32)

    streams = ([(r1s, neg1_ref, i, *stats1[i]) for i in range(ns)]
               + [(r2s, neg2_ref, i, *stats2[i]) for i in range(ns)])
    for refs_s, neg_ref, h, m_scr, l_scr in streams:
        scores = lax.dot_general(
            refs_s[...], neg_ref[h * th:(h + 1) * th, :].astype(jnp.bfloat16),
            dimension_numbers=(((1,), (1,)), ((), ())),
            preferred_element_type=jnp.float32,
        ).astype(jnp.bfloat16)
        _half_update(scores, m_scr, l_scr, th)

    @pl.when(mi == num_mt - 1)
    def _finalize():
        finals = (
            (ref1_ref, pos1_ref, stats1, _INV_TEMPS[0] * _LOG2E, 0),
            (ref2_ref, pos2_ref, stats2, _INV_TEMPS[1] * _LOG2E, 1),
        )
        inv_n = jnp.float32(1.0 / n)
        for ref_ref, pos_ref, stats, scale, k in finals:
            mfs = [ms[...].astype(jnp.float32) for ms, _ in stats]
            m_lane = mfs[0]
            for mf in mfs[1:]:
                m_lane = jnp.maximum(m_lane, mf)
            l_lane = stats[0][1][...] * jnp.exp2(mfs[0] - m_lane)
            for (_, ls), mf in zip(stats[1:], mfs[1:]):
                l_lane = l_lane + ls[...] * jnp.exp2(mf - m_lane)
            m_row = jnp.max(m_lane, axis=-1, keepdims=True)
            l_row = jnp.sum(l_lane * jnp.exp2(m_lane - m_row),
                            axis=-1, keepdims=True)
            pos_dist = jnp.sum(ref_ref[...] * jnp.float32(scale) * pos_ref[...],
                               axis=-1, keepdims=True)
            align = jnp.sum(m_row - pos_dist) * jnp.float32(_LN2) * inv_n
            uniform = jnp.sum(jnp.log(l_row)) * inv_n
            out_ref[k, 0] = align + uniform
            out_ref[k, 1] = align
            out_ref[k, 2] = uniform


def kernel(ref1, pos1, neg1, ref2, pos2, neg2):
    n, d = ref1.shape
    m = neg1.shape[0]
    tm = _TM if m % _TM == 0 else m
    num_mt = m // tm

    row_spec = pl.BlockSpec((n, d), lambda mi: (0, 0))
    neg_spec = pl.BlockSpec((tm, d), lambda mi: (mi, 0))
    ns = _NSTREAM if tm % (_NSTREAM * 128) == 0 else 1
    body = functools.partial(_body, n=n, tm=tm, num_mt=num_mt, ns=ns)
    stat = [pltpu.VMEM((n, 128), jnp.bfloat16),
            pltpu.VMEM((n, 128), jnp.float32)] * (2 * ns)
    return pl.pallas_call(
        body,
        grid=(num_mt,),
        in_specs=[row_spec, row_spec, neg_spec, row_spec, row_spec, neg_spec],
        out_specs=pl.BlockSpec(memory_space=pltpu.MemorySpace.SMEM),
        out_shape=jax.ShapeDtypeStruct((2, 3), jnp.float32),
        scratch_shapes=[
            pltpu.VMEM((n, d), jnp.bfloat16),
            pltpu.VMEM((n, d), jnp.bfloat16),
        ] + stat,
        compiler_params=pltpu.CompilerParams(
            dimension_semantics=("arbitrary",)),
    )(ref1, pos1, neg1, ref2, pos2, neg2)


# speedup vs baseline: 1.0485x; 1.0485x over previous
import functools

import jax
import jax.numpy as jnp
from jax import lax
from jax.experimental import pallas as pl
from jax.experimental.pallas import tpu as pltpu

_LOG2E = 1.4426950408889634
_LN2 = 0.6931471805599453
_INV_TEMPS = (1.0, 2.0)
_TM = 2048
_NSTREAM = 4
_NEG_BIG = -1.0e30


def _half_update(scores, m_scr, l_scr, th):
    chunks = [scores[:, g * 128:(g + 1) * 128] for g in range(th // 128)]
    tile_m = chunks[0]
    for c in chunks[1:]:
        tile_m = jnp.maximum(tile_m, c)
    m_prev = m_scr[...]
    m_new = jnp.maximum(m_prev, tile_m)
    part = jnp.exp2(chunks[0] - m_new)
    for c in chunks[1:]:
        part = part + jnp.exp2(c - m_new)
    alpha = jnp.exp2((m_prev - m_new).astype(jnp.float32))
    l_scr[...] = alpha * l_scr[...] + part.astype(jnp.float32)
    m_scr[...] = m_new


def _body(ref1_ref, pos1_ref, neg1_ref, ref2_ref, pos2_ref, neg2_ref,
          out_ref, r1s, r2s, *scratches, n, tm, num_mt, ns):
    mi = pl.program_id(0)
    th = tm // ns
    stats1 = [(scratches[2 * i], scratches[2 * i + 1]) for i in range(ns)]
    stats2 = [(scratches[2 * ns + 2 * i], scratches[2 * ns + 2 * i + 1])
              for i in range(ns)]

    @pl.when(mi == 0)
    def _init():
        r1s[...] = (ref1_ref[...] * jnp.float32(_INV_TEMPS[0] * _LOG2E)
                    ).astype(jnp.bfloat16)
        r2s[...] = (ref2_ref[...] * jnp.float32(_INV_TEMPS[1] * _LOG2E)
                    ).astype(jnp.bfloat16)
        for m_scr, l_scr in stats1 + stats2:
            m_scr[...] = jnp.full(m_scr.shape, _NEG_BIG, dtype=jnp.bfloat16)
            l_scr[...] = jnp.zeros(l_scr.shape, dtype=jnp.float32)

    streams = ([(r1s, neg1_ref, i, *stats1[i]) for i in range(ns)]
               + [(r2s, neg2_ref, i, *stats2[i]) for i in range(ns)])
    for refs_s, neg_ref, h, m_scr, l_scr in streams:
        scores = lax.dot_general(
            refs_s[...], neg_ref[h * th:(h + 1) * th, :].astype(jnp.bfloat16),
            dimension_numbers=(((1,), (1,)), ((), ())),
            preferred_element_type=jnp.float32,
        ).astype(jnp.bfloat16)
        _half_update(scores, m_scr, l_scr, th)

    @pl.when(mi == num_mt - 1)
    def _finalize():
        finals = (
            (ref1_ref, pos1_ref, stats1, _INV_TEMPS[0] * _LOG2E, 0),
            (ref2_ref, pos2_ref, stats2, _INV_TEMPS[1] * _LOG2E, 1),
        )
        inv_n = jnp.float32(1.0 / n)
        for ref_ref, pos_ref, stats, scale, k in finals:
            mfs = [ms[...].astype(jnp.float32) for ms, _ in stats]
            m_lane = mfs[0]
            for mf in mfs[1:]:
                m_lane = jnp.maximum(m_lane, mf)
            l_lane = stats[0][1][...] * jnp.exp2(mfs[0] - m_lane)
            for (_, ls), mf in zip(stats[1:], mfs[1:]):
                l_lane = l_lane + ls[...] * jnp.exp2(mf - m_lane)
            m_row = jnp.max(m_lane, axis=-1, keepdims=True)
            l_row = jnp.sum(l_lane * jnp.exp2(m_lane - m_row),
                            axis=-1, keepdims=True)
            pos_dist = jnp.sum(ref_ref[...] * jnp.float32(scale) * pos_ref[...],
                               axis=-1, keepdims=True)
            align = jnp.sum(m_row - pos_dist) * jnp.float32(_LN2) * inv_n
            uniform = jnp.sum(jnp.log(l_row)) * inv_n
            out_ref[k, 0] = align + uniform
            out_ref[k, 1] = align
            out_ref[k, 2] = uniform


def kernel(ref1, pos1, neg1, ref2, pos2, neg2):
    n, d = ref1.shape
    m = neg1.shape[0]
    tm = _TM if m % _TM == 0 else m
    num_mt = m // tm

    row_spec = pl.BlockSpec((n, d), lambda mi: (0, 0))
    neg_spec = pl.BlockSpec((tm, d), lambda mi: (mi, 0))
    ns = _NSTREAM if tm % (_NSTREAM * 128) == 0 else 1
    body = functools.partial(_body, n=n, tm=tm, num_mt=num_mt, ns=ns)
    stat = [pltpu.VMEM((n, 128), jnp.bfloat16),
            pltpu.VMEM((n, 128), jnp.float32)] * (2 * ns)
    return pl.pallas_call(
        body,
        grid=(num_mt,),
        in_specs=[row_spec, row_spec, neg_spec, row_spec, row_spec, neg_spec],
        out_specs=pl.BlockSpec(memory_space=pltpu.MemorySpace.SMEM),
        out_shape=jax.ShapeDtypeStruct((2, 3), jnp.float32),
        scratch_shapes=[
            pltpu.VMEM((n, d), jnp.bfloat16),
            pltpu.VMEM((n, d), jnp.bfloat16),
        ] + stat,
        compiler_params=pltpu.CompilerParams(
            dimension_semantics=("arbitrary",)),
    )(ref1, pos1, neg1, ref2, pos2, neg2)
